# baseline (speedup 1.0000x reference)
import jax
import jax.numpy as jnp
from jax import lax
from jax.experimental import pallas as pl
from jax.experimental.pallas import tpu as pltpu, tpu_sc as plsc

N = 10000
NPAD = 10240
E = 320000
D = 128

NC = 2
NS = 16
K = 128
NCHUNK = 80
NG = 5
GCH = NCHUNK // NG
EPAD = NC * NS * NCHUNK * K
ROWS_PER_TILE = NPAD // NS
CNT_W = 16


def _fill_vmem(ref, nrows, ncols, val):
  v = jnp.full((16,), val, jnp.float32)
  def body(i, _):
    for jj in range(ncols // 16):
      ref[i, pl.ds(jj * 16, 16)] = v
    return 0
  lax.fori_loop(0, nrows, body, 0)


def _sc_agg_body(y_hbm, src_hbm, dst_hbm, z_hbm,
                 acc, src_idx, dst_idx, rows, sem0, sem1):
  c = lax.axis_index("c")
  s = lax.axis_index("s")
  base = s * ROWS_PER_TILE

  _fill_vmem(rows.at[0], K, D, 0.0)
  for t in range(ROWS_PER_TILE // K):
    pltpu.sync_copy(rows.at[0], acc.at[pl.ds(base + t * K, K), :])
  if ROWS_PER_TILE % K:
    pltpu.sync_copy(rows.at[0, pl.ds(0, ROWS_PER_TILE % K)],
                    acc.at[pl.ds(base + (ROWS_PER_TILE // K) * K,
                                 ROWS_PER_TILE % K), :])
  plsc.subcore_barrier()

  pltpu.sync_copy(src_hbm.at[c, s], src_idx)

  def group(g, _):
    pltpu.sync_copy(dst_hbm.at[c, s, g], dst_idx)
    cb = g * GCH
    pltpu.async_copy(y_hbm.at[src_idx.at[cb]], rows.at[0], sem0)

    def body(j, _):
      e = cb + 2 * j
      o = e + 1
      pltpu.async_copy(y_hbm.at[src_idx.at[o]], rows.at[1], sem1)
      pltpu.make_async_copy(y_hbm.at[src_idx.at[e]], rows.at[0], sem0).wait()
      pltpu.sync_copy(rows.at[0], acc.at[dst_idx.at[2 * j]], add=True)

      @pl.when(2 * j + 2 < GCH)
      def _():
        pltpu.async_copy(y_hbm.at[src_idx.at[o + 1]], rows.at[0], sem0)

      pltpu.make_async_copy(y_hbm.at[src_idx.at[o]], rows.at[1], sem1).wait()
      pltpu.sync_copy(rows.at[1], acc.at[dst_idx.at[2 * j + 1]], add=True)
      return 0

    lax.fori_loop(0, GCH // 2, body, 0)
    return 0

  lax.fori_loop(0, NG, group, 0)
  plsc.subcore_barrier()

  pltpu.sync_copy(acc.at[pl.ds(base, ROWS_PER_TILE), :],
                  z_hbm.at[c, pl.ds(base, ROWS_PER_TILE), :])


def _sc_cnt_body(dst_hbm, cnt_hbm, cntacc, dst_idx, ones, sem0):
  del sem0
  c = lax.axis_index("c")
  s = lax.axis_index("s")
  base = s * ROWS_PER_TILE

  _fill_vmem(ones, K, D, 0.0)
  for t in range(ROWS_PER_TILE // K):
    pltpu.sync_copy(ones, cntacc.at[pl.ds(base + t * K, K), :])
  if ROWS_PER_TILE % K:
    pltpu.sync_copy(ones.at[pl.ds(0, ROWS_PER_TILE % K)],
                    cntacc.at[pl.ds(base + (ROWS_PER_TILE // K) * K,
                                    ROWS_PER_TILE % K), :])
  _fill_vmem(ones, K, D, 1.0)
  plsc.subcore_barrier()

  def group(g, _):
    pltpu.sync_copy(dst_hbm.at[c, s, g], dst_idx)

    def body(i, _):
      pltpu.sync_copy(ones, cntacc.at[dst_idx.at[i]], add=True)
      return 0

    lax.fori_loop(0, GCH, body, 0)
    return 0

  lax.fori_loop(0, NG, group, 0)
  plsc.subcore_barrier()

  pltpu.sync_copy(cntacc.at[pl.ds(base, ROWS_PER_TILE), :],
                  cnt_hbm.at[c, pl.ds(base, ROWS_PER_TILE), :])


_sc_mesh = plsc.VectorSubcoreMesh(core_axis_name="c", subcore_axis_name="s",
                                  num_cores=NC, num_subcores=NS)

_sc_agg = pl.kernel(
    _sc_agg_body,
    out_type=jax.ShapeDtypeStruct((NC, NPAD, D), jnp.float32),
    mesh=_sc_mesh,
    scratch_types=[
        pltpu.VMEM_SHARED((NPAD, D), jnp.float32),
        pltpu.VMEM((NCHUNK, K), jnp.int32),
        pltpu.VMEM((GCH, K), jnp.int32),
        pltpu.VMEM((2, K, D), jnp.float32),
        pltpu.SemaphoreType.DMA,
        pltpu.SemaphoreType.DMA,
    ],
)

_sc_cnt = pl.kernel(
    _sc_cnt_body,
    out_type=jax.ShapeDtypeStruct((NC, NPAD, D), jnp.float32),
    mesh=_sc_mesh,
    scratch_types=[
        pltpu.VMEM_SHARED((NPAD, D), jnp.float32),
        pltpu.VMEM((GCH, K), jnp.int32),
        pltpu.VMEM((K, D), jnp.float32),
        pltpu.SemaphoreType.DMA,
    ],
)


_BT = 1000


def _tc_in_body(x_ref, wl_ref, wr_ref, b_ref, yl_ref, yr_ref):
  x = x_ref[...]
  yl_ref[...] = jnp.dot(x, wl_ref[...], preferred_element_type=jnp.float32)
  yr_ref[...] = (jnp.dot(x, wr_ref[...], preferred_element_type=jnp.float32)
                 + b_ref[...])


def _combine(z_ref, cnt_ref, yr_ref, a_ref):
  zblk = z_ref[...]
  cblk = cnt_ref[...]
  cnt = cblk[0, :, 0:1] + cblk[1, :, 0:1]
  agg = (zblk[0] + zblk[1]) / jnp.maximum(cnt, 1.0)
  h = agg + yr_ref[...]
  return jnp.where(h >= 0.0, h, a_ref[...] * h)


def _tc_mid_body(z_ref, cnt_ref, yr_ref, a_ref, wl_ref, wr_ref, b_ref,
                 yl_out, yr_out):
  h = _combine(z_ref, cnt_ref, yr_ref, a_ref)
  yl_out[...] = jnp.dot(h, wl_ref[...], preferred_element_type=jnp.float32)
  yr_out[...] = (jnp.dot(h, wr_ref[...], preferred_element_type=jnp.float32)
                 + b_ref[...])


def _tc_out_body(z_ref, cnt_ref, yr_ref, a_ref, h_out):
  h_out[...] = _combine(z_ref, cnt_ref, yr_ref, a_ref)


_row_spec = pl.BlockSpec((_BT, D), lambda i: (i, 0))
_w_spec = pl.BlockSpec((D, D), lambda i: (0, 0))
_v_spec = pl.BlockSpec((1, D), lambda i: (0, 0))
_z_spec = pl.BlockSpec((NC, _BT, D), lambda i: (0, i, 0))
_c_spec = pl.BlockSpec((NC, _BT, CNT_W), lambda i: (0, i, 0))

_tc_in = pl.pallas_call(
    _tc_in_body,
    grid=(N // _BT,),
    in_specs=[_row_spec, _w_spec, _w_spec, _v_spec],
    out_specs=[_row_spec, _row_spec],
    out_shape=[jax.ShapeDtypeStruct((N, D), jnp.float32)] * 2,
)

_tc_mid = pl.pallas_call(
    _tc_mid_body,
    grid=(N // _BT,),
    in_specs=[_z_spec, _c_spec, _row_spec, _v_spec, _w_spec, _w_spec, _v_spec],
    out_specs=[_row_spec, _row_spec],
    out_shape=[jax.ShapeDtypeStruct((N, D), jnp.float32)] * 2,
)

_tc_out = pl.pallas_call(
    _tc_out_body,
    grid=(N // _BT,),
    in_specs=[_z_spec, _c_spec, _row_spec, _v_spec],
    out_specs=_row_spec,
    out_shape=jax.ShapeDtypeStruct((N, D), jnp.float32),
)


def kernel(x, edge_index, W1_l, W1_r, b1, a1, W2_l, W2_r, b2, a2,
           W3_l, W3_r, b3, a3):
  pad_src = jnp.zeros((EPAD - E,), jnp.int32)
  pad_dst = jnp.full((EPAD - E,), N, jnp.int32)
  src = jnp.concatenate([edge_index[0].astype(jnp.int32), pad_src])
  dst = jnp.concatenate([edge_index[1].astype(jnp.int32), pad_dst])
  src = src.reshape(NC, NS, NCHUNK, K)
  dst = dst.reshape(NC, NS, NG, GCH, K)
  b1r = b1.reshape(1, D)
  b2r = b2.reshape(1, D)
  b3r = b3.reshape(1, D)
  a1r = a1.reshape(1, D)
  a2r = a2.reshape(1, D)
  a3r = a3.reshape(1, D)

  cnt = _sc_cnt(dst)[:, :, :CNT_W]
  y1l, y1r = _tc_in(x, W1_l, W1_r, b1r)
  z1 = _sc_agg(y1l, src, dst)
  y2l, y2r = _tc_mid(z1, cnt, y1r, a1r, W2_l, W2_r, b2r)
  z2 = _sc_agg(y2l, src, dst)
  y3l, y3r = _tc_mid(z2, cnt, y2r, a2r, W3_l, W3_r, b3r)
  z3 = _sc_agg(y3l, src, dst)
  return _tc_out(z3, cnt, y3r, a3r)


# speedup vs baseline: 2.6047x; 2.6047x over previous
import jax
import jax.numpy as jnp
from jax import lax
from jax.experimental import pallas as pl
from jax.experimental.pallas import tpu as pltpu, tpu_sc as plsc

N = 10000
NPAD = 10240
E = 320000
D = 128

NC = 2
NS = 16
K = 100
NCHUNK = 100
NG = 5
GCH = NCHUNK // NG
EPAD = NC * NS * NCHUNK * K
ROWS_PER_TILE = NPAD // NS
CNT_W = 16


def _fill_vmem(ref, nrows, ncols, val):
  v = jnp.full((16,), val, jnp.float32)
  def body(i, _):
    for jj in range(ncols // 16):
      ref[i, pl.ds(jj * 16, 16)] = v
    return 0
  lax.fori_loop(0, nrows, body, 0)


def _sc_agg_body(y_hbm, src_hbm, dst_hbm, z_hbm,
                 acc, src_idx, dst_idx, rows, sem0, sem1, sem2, sem3):
  c = lax.axis_index("c")
  s = lax.axis_index("s")
  base = s * ROWS_PER_TILE

  _fill_vmem(rows.at[0], K, D, 0.0)
  for t in range(ROWS_PER_TILE // K):
    pltpu.sync_copy(rows.at[0], acc.at[pl.ds(base + t * K, K), :])
  if ROWS_PER_TILE % K:
    pltpu.sync_copy(rows.at[0, pl.ds(0, ROWS_PER_TILE % K)],
                    acc.at[pl.ds(base + (ROWS_PER_TILE // K) * K,
                                 ROWS_PER_TILE % K), :])
  plsc.subcore_barrier()

  pltpu.sync_copy(src_hbm.at[c, s], src_idx)

  def group(g, _):
    pltpu.sync_copy(dst_hbm.at[c, s, g], dst_idx)
    cb = g * GCH
    pltpu.async_copy(y_hbm.at[src_idx.at[cb]], rows.at[0], sem0)

    def body(j, _):
      e = cb + 2 * j
      o = e + 1
      pltpu.async_copy(y_hbm.at[src_idx.at[o]], rows.at[1], sem1)
      pltpu.make_async_copy(y_hbm.at[src_idx.at[e]], rows.at[0], sem0).wait()
      pltpu.async_copy(rows.at[0], acc.at[dst_idx.at[2 * j]], sem2, add=True)
      pltpu.make_async_copy(y_hbm.at[src_idx.at[o]], rows.at[1], sem1).wait()
      pltpu.async_copy(rows.at[1], acc.at[dst_idx.at[2 * j + 1]], sem3,
                       add=True)
      pltpu.make_async_copy(rows.at[0], acc.at[dst_idx.at[2 * j]],
                            sem2).wait()

      @pl.when(2 * j + 2 < GCH)
      def _():
        pltpu.async_copy(y_hbm.at[src_idx.at[o + 1]], rows.at[0], sem0)

      pltpu.make_async_copy(rows.at[1], acc.at[dst_idx.at[2 * j + 1]],
                            sem3).wait()
      return 0

    lax.fori_loop(0, GCH // 2, body, 0)
    return 0

  lax.fori_loop(0, NG, group, 0)
  plsc.subcore_barrier()

  pltpu.sync_copy(acc.at[pl.ds(base, ROWS_PER_TILE), :],
                  z_hbm.at[c, pl.ds(base, ROWS_PER_TILE), :])


def _sc_cnt_body(dst_hbm, cnt_hbm, cntacc, dst_idx, ones, sem0):
  del sem0
  c = lax.axis_index("c")
  s = lax.axis_index("s")
  base = s * ROWS_PER_TILE

  _fill_vmem(ones, K, D, 0.0)
  for t in range(ROWS_PER_TILE // K):
    pltpu.sync_copy(ones, cntacc.at[pl.ds(base + t * K, K), :])
  if ROWS_PER_TILE % K:
    pltpu.sync_copy(ones.at[pl.ds(0, ROWS_PER_TILE % K)],
                    cntacc.at[pl.ds(base + (ROWS_PER_TILE // K) * K,
                                    ROWS_PER_TILE % K), :])
  _fill_vmem(ones, K, D, 1.0)
  plsc.subcore_barrier()

  def group(g, _):
    pltpu.sync_copy(dst_hbm.at[c, s, g], dst_idx)

    def body(i, _):
      pltpu.sync_copy(ones, cntacc.at[dst_idx.at[i]], add=True)
      return 0

    lax.fori_loop(0, GCH, body, 0)
    return 0

  lax.fori_loop(0, NG, group, 0)
  plsc.subcore_barrier()

  pltpu.sync_copy(cntacc.at[pl.ds(base, ROWS_PER_TILE), :],
                  cnt_hbm.at[c, pl.ds(base, ROWS_PER_TILE), :])


_sc_mesh = plsc.VectorSubcoreMesh(core_axis_name="c", subcore_axis_name="s",
                                  num_cores=NC, num_subcores=NS)

_sc_agg = pl.kernel(
    _sc_agg_body,
    out_type=jax.ShapeDtypeStruct((NC, NPAD, D), jnp.float32),
    mesh=_sc_mesh,
    scratch_types=[
        pltpu.VMEM_SHARED((NPAD, D), jnp.float32),
        pltpu.VMEM((NCHUNK, K), jnp.int32),
        pltpu.VMEM((GCH, K), jnp.int32),
        pltpu.VMEM((2, K, D), jnp.float32),
        pltpu.SemaphoreType.DMA,
        pltpu.SemaphoreType.DMA,
        pltpu.SemaphoreType.DMA,
        pltpu.SemaphoreType.DMA,
    ],
)

_sc_cnt = pl.kernel(
    _sc_cnt_body,
    out_type=jax.ShapeDtypeStruct((NC, NPAD, D), jnp.float32),
    mesh=_sc_mesh,
    scratch_types=[
        pltpu.VMEM_SHARED((NPAD, D), jnp.float32),
        pltpu.VMEM((GCH, K), jnp.int32),
        pltpu.VMEM((K, D), jnp.float32),
        pltpu.SemaphoreType.DMA,
    ],
)


_BT = 1000


def _tc_in_body(x_ref, wl_ref, wr_ref, b_ref, yl_ref, yr_ref):
  x = x_ref[...]
  yl_ref[...] = jnp.dot(x, wl_ref[...], preferred_element_type=jnp.float32)
  yr_ref[...] = (jnp.dot(x, wr_ref[...], preferred_element_type=jnp.float32)
                 + b_ref[...])


def _combine(z_ref, cnt_ref, yr_ref, a_ref):
  zblk = z_ref[...]
  cblk = cnt_ref[...]
  cnt = cblk[0, :, 0:1] + cblk[1, :, 0:1]
  agg = (zblk[0] + zblk[1]) / jnp.maximum(cnt, 1.0)
  h = agg + yr_ref[...]
  return jnp.where(h >= 0.0, h, a_ref[...] * h)


def _tc_mid_body(z_ref, cnt_ref, yr_ref, a_ref, wl_ref, wr_ref, b_ref,
                 yl_out, yr_out):
  h = _combine(z_ref, cnt_ref, yr_ref, a_ref)
  yl_out[...] = jnp.dot(h, wl_ref[...], preferred_element_type=jnp.float32)
  yr_out[...] = (jnp.dot(h, wr_ref[...], preferred_element_type=jnp.float32)
                 + b_ref[...])


def _tc_out_body(z_ref, cnt_ref, yr_ref, a_ref, h_out):
  h_out[...] = _combine(z_ref, cnt_ref, yr_ref, a_ref)


_row_spec = pl.BlockSpec((_BT, D), lambda i: (i, 0))
_w_spec = pl.BlockSpec((D, D), lambda i: (0, 0))
_v_spec = pl.BlockSpec((1, D), lambda i: (0, 0))
_z_spec = pl.BlockSpec((NC, _BT, D), lambda i: (0, i, 0))
_c_spec = pl.BlockSpec((NC, _BT, CNT_W), lambda i: (0, i, 0))

_tc_in = pl.pallas_call(
    _tc_in_body,
    grid=(N // _BT,),
    in_specs=[_row_spec, _w_spec, _w_spec, _v_spec],
    out_specs=[_row_spec, _row_spec],
    out_shape=[jax.ShapeDtypeStruct((N, D), jnp.float32)] * 2,
)

_tc_mid = pl.pallas_call(
    _tc_mid_body,
    grid=(N // _BT,),
    in_specs=[_z_spec, _c_spec, _row_spec, _v_spec, _w_spec, _w_spec, _v_spec],
    out_specs=[_row_spec, _row_spec],
    out_shape=[jax.ShapeDtypeStruct((N, D), jnp.float32)] * 2,
)

_tc_out = pl.pallas_call(
    _tc_out_body,
    grid=(N // _BT,),
    in_specs=[_z_spec, _c_spec, _row_spec, _v_spec],
    out_specs=_row_spec,
    out_shape=jax.ShapeDtypeStruct((N, D), jnp.float32),
)


def kernel(x, edge_index, W1_l, W1_r, b1, a1, W2_l, W2_r, b2, a2,
           W3_l, W3_r, b3, a3):
  src = edge_index[0].astype(jnp.int32).reshape(NC, NS, NCHUNK, K)
  dst = edge_index[1].astype(jnp.int32).reshape(NC, NS, NG, GCH, K)
  b1r = b1.reshape(1, D)
  b2r = b2.reshape(1, D)
  b3r = b3.reshape(1, D)
  a1r = a1.reshape(1, D)
  a2r = a2.reshape(1, D)
  a3r = a3.reshape(1, D)

  cnt = _sc_cnt(dst)[:, :, :CNT_W]
  y1l, y1r = _tc_in(x, W1_l, W1_r, b1r)
  z1 = _sc_agg(y1l, src, dst)
  y2l, y2r = _tc_mid(z1, cnt, y1r, a1r, W2_l, W2_r, b2r)
  z2 = _sc_agg(y2l, src, dst)
  y3l, y3r = _tc_mid(z2, cnt, y2r, a2r, W3_l, W3_r, b3r)
  z3 = _sc_agg(y3l, src, dst)
  return _tc_out(z3, cnt, y3r, a3r)


# speedup vs baseline: 3.2201x; 1.2363x over previous
import jax
import jax.numpy as jnp
from jax import lax
from jax.experimental import pallas as pl
from jax.experimental.pallas import tpu as pltpu, tpu_sc as plsc

N = 10000
NPAD = 10240
E = 320000
D = 128

NC = 2
NS = 16
K = 100
NCHUNK = 100
NG = 2
GCH = NCHUNK // NG
EPAD = NC * NS * NCHUNK * K
ROWS_PER_TILE = NPAD // NS
CNT_W = 16


def _fill_vmem(ref, nrows, ncols, val):
  v = jnp.full((16,), val, jnp.float32)
  def body(i, _):
    for jj in range(ncols // 16):
      ref[i, pl.ds(jj * 16, 16)] = v
    return 0
  lax.fori_loop(0, nrows, body, 0)


def _sc_agg_body(y_hbm, src_hbm, dst_hbm, z_hbm,
                 acc, src_idx, dst_idx, rows, sem0, sem1):
  c = lax.axis_index("c")
  s = lax.axis_index("s")
  base = s * ROWS_PER_TILE

  _fill_vmem(rows.at[0], K, D, 0.0)
  for t in range(ROWS_PER_TILE // K):
    pltpu.sync_copy(rows.at[0], acc.at[pl.ds(base + t * K, K), :])
  if ROWS_PER_TILE % K:
    pltpu.sync_copy(rows.at[0, pl.ds(0, ROWS_PER_TILE % K)],
                    acc.at[pl.ds(base + (ROWS_PER_TILE // K) * K,
                                 ROWS_PER_TILE % K), :])
  plsc.subcore_barrier()

  pltpu.sync_copy(src_hbm.at[c, s], src_idx)

  def group(g, _):
    pltpu.sync_copy(dst_hbm.at[c, s, g], dst_idx)
    cb = g * GCH
    pltpu.async_copy(y_hbm.at[src_idx.at[cb]], rows.at[0], sem0)

    def body(j, _):
      e = cb + 2 * j
      o = e + 1
      pltpu.async_copy(y_hbm.at[src_idx.at[o]], rows.at[1], sem1)
      pltpu.make_async_copy(y_hbm.at[src_idx.at[e]], rows.at[0], sem0).wait()
      pltpu.sync_copy(rows.at[0], acc.at[dst_idx.at[2 * j]], add=True)

      @pl.when(2 * j + 2 < GCH)
      def _():
        pltpu.async_copy(y_hbm.at[src_idx.at[o + 1]], rows.at[0], sem0)

      pltpu.make_async_copy(y_hbm.at[src_idx.at[o]], rows.at[1], sem1).wait()
      pltpu.sync_copy(rows.at[1], acc.at[dst_idx.at[2 * j + 1]], add=True)
      return 0

    lax.fori_loop(0, GCH // 2, body, 0)
    return 0

  lax.fori_loop(0, NG, group, 0)
  plsc.subcore_barrier()

  pltpu.sync_copy(acc.at[pl.ds(base, ROWS_PER_TILE), :],
                  z_hbm.at[c, pl.ds(base, ROWS_PER_TILE), :])


def _sc_cnt_body(dst_hbm, cnt_hbm, cntacc, dst_idx, ones, sem0):
  del sem0
  c = lax.axis_index("c")
  s = lax.axis_index("s")
  base = s * ROWS_PER_TILE

  _fill_vmem(ones, K, D, 0.0)
  for t in range(ROWS_PER_TILE // K):
    pltpu.sync_copy(ones, cntacc.at[pl.ds(base + t * K, K), :])
  if ROWS_PER_TILE % K:
    pltpu.sync_copy(ones.at[pl.ds(0, ROWS_PER_TILE % K)],
                    cntacc.at[pl.ds(base + (ROWS_PER_TILE // K) * K,
                                    ROWS_PER_TILE % K), :])
  _fill_vmem(ones, K, D, 1.0)
  plsc.subcore_barrier()

  def group(g, _):
    pltpu.sync_copy(dst_hbm.at[c, s, g], dst_idx)

    def body(i, _):
      pltpu.sync_copy(ones, cntacc.at[dst_idx.at[i]], add=True)
      return 0

    lax.fori_loop(0, GCH, body, 0)
    return 0

  lax.fori_loop(0, NG, group, 0)
  plsc.subcore_barrier()

  pltpu.sync_copy(cntacc.at[pl.ds(base, ROWS_PER_TILE), :],
                  cnt_hbm.at[c, pl.ds(base, ROWS_PER_TILE), :])


_sc_mesh = plsc.VectorSubcoreMesh(core_axis_name="c", subcore_axis_name="s",
                                  num_cores=NC, num_subcores=NS)

_sc_agg = pl.kernel(
    _sc_agg_body,
    out_type=jax.ShapeDtypeStruct((NC, NPAD, D), jnp.float32),
    mesh=_sc_mesh,
    scratch_types=[
        pltpu.VMEM_SHARED((NPAD, D), jnp.float32),
        pltpu.VMEM((NCHUNK, K), jnp.int32),
        pltpu.VMEM((GCH, K), jnp.int32),
        pltpu.VMEM((2, K, D), jnp.float32),
        pltpu.SemaphoreType.DMA,
        pltpu.SemaphoreType.DMA,
    ],
)

_sc_cnt = pl.kernel(
    _sc_cnt_body,
    out_type=jax.ShapeDtypeStruct((NC, NPAD, D), jnp.float32),
    mesh=_sc_mesh,
    scratch_types=[
        pltpu.VMEM_SHARED((NPAD, D), jnp.float32),
        pltpu.VMEM((GCH, K), jnp.int32),
        pltpu.VMEM((K, D), jnp.float32),
        pltpu.SemaphoreType.DMA,
    ],
)


_BT = 1000


def _tc_in_body(x_ref, wl_ref, wr_ref, b_ref, yl_ref, yr_ref):
  x = x_ref[...]
  yl_ref[...] = jnp.dot(x, wl_ref[...], preferred_element_type=jnp.float32)
  yr_ref[...] = (jnp.dot(x, wr_ref[...], preferred_element_type=jnp.float32)
                 + b_ref[...])


def _combine(z_ref, cnt_ref, yr_ref, a_ref):
  zblk = z_ref[...]
  cblk = cnt_ref[...]
  cnt = cblk[0, :, 0:1] + cblk[1, :, 0:1]
  agg = (zblk[0] + zblk[1]) / jnp.maximum(cnt, 1.0)
  h = agg + yr_ref[...]
  return jnp.where(h >= 0.0, h, a_ref[...] * h)


def _tc_mid_body(z_ref, cnt_ref, yr_ref, a_ref, wl_ref, wr_ref, b_ref,
                 yl_out, yr_out):
  h = _combine(z_ref, cnt_ref, yr_ref, a_ref)
  yl_out[...] = jnp.dot(h, wl_ref[...], preferred_element_type=jnp.float32)
  yr_out[...] = (jnp.dot(h, wr_ref[...], preferred_element_type=jnp.float32)
                 + b_ref[...])


def _tc_out_body(z_ref, cnt_ref, yr_ref, a_ref, h_out):
  h_out[...] = _combine(z_ref, cnt_ref, yr_ref, a_ref)


_row_spec = pl.BlockSpec((_BT, D), lambda i: (i, 0))
_w_spec = pl.BlockSpec((D, D), lambda i: (0, 0))
_v_spec = pl.BlockSpec((1, D), lambda i: (0, 0))
_z_spec = pl.BlockSpec((NC, _BT, D), lambda i: (0, i, 0))
_c_spec = pl.BlockSpec((NC, _BT, CNT_W), lambda i: (0, i, 0))

_tc_in = pl.pallas_call(
    _tc_in_body,
    grid=(N // _BT,),
    in_specs=[_row_spec, _w_spec, _w_spec, _v_spec],
    out_specs=[_row_spec, _row_spec],
    out_shape=[jax.ShapeDtypeStruct((N, D), jnp.float32)] * 2,
)

_tc_mid = pl.pallas_call(
    _tc_mid_body,
    grid=(N // _BT,),
    in_specs=[_z_spec, _c_spec, _row_spec, _v_spec, _w_spec, _w_spec, _v_spec],
    out_specs=[_row_spec, _row_spec],
    out_shape=[jax.ShapeDtypeStruct((N, D), jnp.float32)] * 2,
)

_tc_out = pl.pallas_call(
    _tc_out_body,
    grid=(N // _BT,),
    in_specs=[_z_spec, _c_spec, _row_spec, _v_spec],
    out_specs=_row_spec,
    out_shape=jax.ShapeDtypeStruct((N, D), jnp.float32),
)


def kernel(x, edge_index, W1_l, W1_r, b1, a1, W2_l, W2_r, b2, a2,
           W3_l, W3_r, b3, a3):
  src = edge_index[0].astype(jnp.int32).reshape(NC, NS, NCHUNK, K)
  dst = edge_index[1].astype(jnp.int32).reshape(NC, NS, NG, GCH, K)
  b1r = b1.reshape(1, D)
  b2r = b2.reshape(1, D)
  b3r = b3.reshape(1, D)
  a1r = a1.reshape(1, D)
  a2r = a2.reshape(1, D)
  a3r = a3.reshape(1, D)

  cnt = _sc_cnt(dst)[:, :, :CNT_W]
  y1l, y1r = _tc_in(x, W1_l, W1_r, b1r)
  z1 = _sc_agg(y1l, src, dst)
  y2l, y2r = _tc_mid(z1, cnt, y1r, a1r, W2_l, W2_r, b2r)
  z2 = _sc_agg(y2l, src, dst)
  y3l, y3r = _tc_mid(z2, cnt, y2r, a2r, W3_l, W3_r, b3r)
  z3 = _sc_agg(y3l, src, dst)
  return _tc_out(z3, cnt, y3r, a3r)


# speedup vs baseline: 3.2928x; 1.0226x over previous
import jax
import jax.numpy as jnp
from jax import lax
from jax.experimental import pallas as pl
from jax.experimental.pallas import tpu as pltpu, tpu_sc as plsc

N = 10000
NPAD = 10240
E = 320000
D = 128

NC = 2
NS = 16
K = 125
NCHUNK = 80
NG = 2
GCH = NCHUNK // NG
EPAD = NC * NS * NCHUNK * K
ROWS_PER_TILE = NPAD // NS
CNT_W = 16


def _fill_vmem(ref, nrows, ncols, val):
  v = jnp.full((16,), val, jnp.float32)
  def body(i, _):
    for jj in range(ncols // 16):
      ref[i, pl.ds(jj * 16, 16)] = v
    return 0
  lax.fori_loop(0, nrows, body, 0)


def _sc_agg_body(y_hbm, src_hbm, dst_hbm, z_hbm,
                 acc, src_idx, dst_idx, rows, sem0, sem1):
  c = lax.axis_index("c")
  s = lax.axis_index("s")
  base = s * ROWS_PER_TILE

  _fill_vmem(rows.at[0], K, D, 0.0)
  for t in range(ROWS_PER_TILE // K):
    pltpu.sync_copy(rows.at[0], acc.at[pl.ds(base + t * K, K), :])
  if ROWS_PER_TILE % K:
    pltpu.sync_copy(rows.at[0, pl.ds(0, ROWS_PER_TILE % K)],
                    acc.at[pl.ds(base + (ROWS_PER_TILE // K) * K,
                                 ROWS_PER_TILE % K), :])
  plsc.subcore_barrier()

  pltpu.sync_copy(src_hbm.at[c, s], src_idx)

  def group(g, _):
    pltpu.sync_copy(dst_hbm.at[c, s, g], dst_idx)
    cb = g * GCH
    pltpu.async_copy(y_hbm.at[src_idx.at[cb]], rows.at[0], sem0)

    def body(j, _):
      e = cb + 2 * j
      o = e + 1
      pltpu.async_copy(y_hbm.at[src_idx.at[o]], rows.at[1], sem1)
      pltpu.make_async_copy(y_hbm.at[src_idx.at[e]], rows.at[0], sem0).wait()
      pltpu.sync_copy(rows.at[0], acc.at[dst_idx.at[2 * j]], add=True)

      @pl.when(2 * j + 2 < GCH)
      def _():
        pltpu.async_copy(y_hbm.at[src_idx.at[o + 1]], rows.at[0], sem0)

      pltpu.make_async_copy(y_hbm.at[src_idx.at[o]], rows.at[1], sem1).wait()
      pltpu.sync_copy(rows.at[1], acc.at[dst_idx.at[2 * j + 1]], add=True)
      return 0

    lax.fori_loop(0, GCH // 2, body, 0)
    return 0

  lax.fori_loop(0, NG, group, 0)
  plsc.subcore_barrier()

  pltpu.sync_copy(acc.at[pl.ds(base, ROWS_PER_TILE), :],
                  z_hbm.at[c, pl.ds(base, ROWS_PER_TILE), :])


def _sc_cnt_body(dst_hbm, cnt_hbm, cntacc, dst_idx, ones, sem0):
  del sem0
  c = lax.axis_index("c")
  s = lax.axis_index("s")
  base = s * ROWS_PER_TILE

  _fill_vmem(ones, K, D, 0.0)
  for t in range(ROWS_PER_TILE // K):
    pltpu.sync_copy(ones, cntacc.at[pl.ds(base + t * K, K), :])
  if ROWS_PER_TILE % K:
    pltpu.sync_copy(ones.at[pl.ds(0, ROWS_PER_TILE % K)],
                    cntacc.at[pl.ds(base + (ROWS_PER_TILE // K) * K,
                                    ROWS_PER_TILE % K), :])
  _fill_vmem(ones, K, D, 1.0)
  plsc.subcore_barrier()

  def group(g, _):
    pltpu.sync_copy(dst_hbm.at[c, s, g], dst_idx)

    def body(i, _):
      pltpu.sync_copy(ones, cntacc.at[dst_idx.at[i]], add=True)
      return 0

    lax.fori_loop(0, GCH, body, 0)
    return 0

  lax.fori_loop(0, NG, group, 0)
  plsc.subcore_barrier()

  pltpu.sync_copy(cntacc.at[pl.ds(base, ROWS_PER_TILE), :],
                  cnt_hbm.at[c, pl.ds(base, ROWS_PER_TILE), :])


_sc_mesh = plsc.VectorSubcoreMesh(core_axis_name="c", subcore_axis_name="s",
                                  num_cores=NC, num_subcores=NS)

_sc_agg = pl.kernel(
    _sc_agg_body,
    out_type=jax.ShapeDtypeStruct((NC, NPAD, D), jnp.float32),
    mesh=_sc_mesh,
    scratch_types=[
        pltpu.VMEM_SHARED((NPAD, D), jnp.float32),
        pltpu.VMEM((NCHUNK, K), jnp.int32),
        pltpu.VMEM((GCH, K), jnp.int32),
        pltpu.VMEM((2, K, D), jnp.float32),
        pltpu.SemaphoreType.DMA,
        pltpu.SemaphoreType.DMA,
    ],
)

_sc_cnt = pl.kernel(
    _sc_cnt_body,
    out_type=jax.ShapeDtypeStruct((NC, NPAD, D), jnp.float32),
    mesh=_sc_mesh,
    scratch_types=[
        pltpu.VMEM_SHARED((NPAD, D), jnp.float32),
        pltpu.VMEM((GCH, K), jnp.int32),
        pltpu.VMEM((K, D), jnp.float32),
        pltpu.SemaphoreType.DMA,
    ],
)


_BT = 1000


def _tc_in_body(x_ref, wl_ref, wr_ref, b_ref, yl_ref, yr_ref):
  x = x_ref[...]
  yl_ref[...] = jnp.dot(x, wl_ref[...], preferred_element_type=jnp.float32)
  yr_ref[...] = (jnp.dot(x, wr_ref[...], preferred_element_type=jnp.float32)
                 + b_ref[...])


def _combine(z_ref, cnt_ref, yr_ref, a_ref):
  zblk = z_ref[...]
  cblk = cnt_ref[...]
  cnt = cblk[0, :, 0:1] + cblk[1, :, 0:1]
  agg = (zblk[0] + zblk[1]) / jnp.maximum(cnt, 1.0)
  h = agg + yr_ref[...]
  return jnp.where(h >= 0.0, h, a_ref[...] * h)


def _tc_mid_body(z_ref, cnt_ref, yr_ref, a_ref, wl_ref, wr_ref, b_ref,
                 yl_out, yr_out):
  h = _combine(z_ref, cnt_ref, yr_ref, a_ref)
  yl_out[...] = jnp.dot(h, wl_ref[...], preferred_element_type=jnp.float32)
  yr_out[...] = (jnp.dot(h, wr_ref[...], preferred_element_type=jnp.float32)
                 + b_ref[...])


def _tc_out_body(z_ref, cnt_ref, yr_ref, a_ref, h_out):
  h_out[...] = _combine(z_ref, cnt_ref, yr_ref, a_ref)


_row_spec = pl.BlockSpec((_BT, D), lambda i: (i, 0))
_w_spec = pl.BlockSpec((D, D), lambda i: (0, 0))
_v_spec = pl.BlockSpec((1, D), lambda i: (0, 0))
_z_spec = pl.BlockSpec((NC, _BT, D), lambda i: (0, i, 0))
_c_spec = pl.BlockSpec((NC, _BT, CNT_W), lambda i: (0, i, 0))

_tc_in = pl.pallas_call(
    _tc_in_body,
    grid=(N // _BT,),
    in_specs=[_row_spec, _w_spec, _w_spec, _v_spec],
    out_specs=[_row_spec, _row_spec],
    out_shape=[jax.ShapeDtypeStruct((N, D), jnp.float32)] * 2,
)

_tc_mid = pl.pallas_call(
    _tc_mid_body,
    grid=(N // _BT,),
    in_specs=[_z_spec, _c_spec, _row_spec, _v_spec, _w_spec, _w_spec, _v_spec],
    out_specs=[_row_spec, _row_spec],
    out_shape=[jax.ShapeDtypeStruct((N, D), jnp.float32)] * 2,
)

_tc_out = pl.pallas_call(
    _tc_out_body,
    grid=(N // _BT,),
    in_specs=[_z_spec, _c_spec, _row_spec, _v_spec],
    out_specs=_row_spec,
    out_shape=jax.ShapeDtypeStruct((N, D), jnp.float32),
)


def kernel(x, edge_index, W1_l, W1_r, b1, a1, W2_l, W2_r, b2, a2,
           W3_l, W3_r, b3, a3):
  src = edge_index[0].astype(jnp.int32).reshape(NC, NS, NCHUNK, K)
  dst = edge_index[1].astype(jnp.int32).reshape(NC, NS, NG, GCH, K)
  b1r = b1.reshape(1, D)
  b2r = b2.reshape(1, D)
  b3r = b3.reshape(1, D)
  a1r = a1.reshape(1, D)
  a2r = a2.reshape(1, D)
  a3r = a3.reshape(1, D)

  cnt = _sc_cnt(dst)[:, :, :CNT_W]
  y1l, y1r = _tc_in(x, W1_l, W1_r, b1r)
  z1 = _sc_agg(y1l, src, dst)
  y2l, y2r = _tc_mid(z1, cnt, y1r, a1r, W2_l, W2_r, b2r)
  z2 = _sc_agg(y2l, src, dst)
  y3l, y3r = _tc_mid(z2, cnt, y2r, a2r, W3_l, W3_r, b3r)
  z3 = _sc_agg(y3l, src, dst)
  return _tc_out(z3, cnt, y3r, a3r)
